# baseline (speedup 1.0000x reference)
import functools

import jax
import jax.numpy as jnp
from jax import lax
from jax.experimental import pallas as pl
from jax.experimental.pallas import tpu as pltpu
from jax.experimental.pallas import tpu_sc as plsc

_NC = 2
_NS = 16
_LN = 16


def _hist_sizes(E: int, K: int, NBINS: int):
    NW = _NC * _NS
    ROWS = E // 128
    RB = ROWS // NW
    EXTRA = ROWS - RB * NW
    KR = RB + (1 if EXTRA else 0)
    HSIZE = -(-NBINS // (_NS * 128)) * (_NS * 128)
    return ROWS, RB, EXTRA, KR, HSIZE


@functools.lru_cache(maxsize=None)
def _build_hist_kernel(E: int, K: int, NBINS: int):
    NW = _NC * _NS
    R = K // 2
    ROWS, RB, EXTRA, KR, HSIZE = _hist_sizes(E, K, NBINS)
    EPW = KR * 128
    CH = HSIZE // _NS

    mesh = plsc.VectorSubcoreMesh(
        core_axis_name="c", subcore_axis_name="s",
        num_cores=_NC, num_subcores=_NS)

    @functools.partial(
        pl.kernel,
        out_type=jax.ShapeDtypeStruct((_NC * HSIZE,), jnp.float32),
        mesh=mesh,
        scratch_types=[
            pltpu.VMEM((EPW,), jnp.int32),
            pltpu.VMEM((EPW,), jnp.int32),
            pltpu.VMEM((2 * KR, 128), jnp.int32),
            pltpu.VMEM((RB, 128), jnp.float32),
            pltpu.VMEM_SHARED((HSIZE,), jnp.float32),
            pltpu.SemaphoreType.DMA,
        ],
    )
    def hist_kernel(ei_hbm, et_hbm, zeros_hbm, ones_hbm, out_hbm,
                    tbuf, ebuf, keys, ones_v, hist_sh, ssem):
        cid = lax.axis_index("c")
        sid = lax.axis_index("s")
        wid = cid * _NS + sid
        nrows = RB + jnp.where(wid < EXTRA, 1, 0)
        row0 = wid * RB + jnp.minimum(wid, EXTRA)
        base = pl.multiple_of(row0 * 128, 128)
        zoff = pl.multiple_of(sid * CH, 128)

        pltpu.sync_copy(zeros_hbm, hist_sh.at[pl.ds(zoff, CH)])
        pltpu.sync_copy(ones_hbm, ones_v)
        pltpu.sync_copy(et_hbm.at[pl.ds(base, RB * 128)], tbuf.at[pl.ds(0, RB * 128)])
        if EXTRA:
            @pl.when(wid < EXTRA)
            def _():
                off = pl.multiple_of(base + RB * 128, 128)
                pltpu.sync_copy(et_hbm.at[pl.ds(off, 128)],
                                tbuf.at[pl.ds(RB * 128, 128)])

        for d in (0, 1):
            dbase = pl.multiple_of(d * E + base, 128)
            pltpu.sync_copy(ei_hbm.at[pl.ds(dbase, RB * 128)],
                            ebuf.at[pl.ds(0, RB * 128)])
            if EXTRA:
                @pl.when(wid < EXTRA)
                def _(d=d, dbase=dbase):
                    off = pl.multiple_of(dbase + RB * 128, 128)
                    pltpu.sync_copy(ei_hbm.at[pl.ds(off, 128)],
                                    ebuf.at[pl.ds(RB * 128, 128)])

            def row_body(j, _, d=d):
                for c in range(8):
                    off = pl.multiple_of(j * 128 + c * _LN, 8)
                    vi = ebuf[pl.ds(off, _LN)]
                    vt = tbuf[pl.ds(off, _LN)]
                    keys[d * KR + j, pl.ds(c * _LN, _LN)] = vi * K + vt + d * R
                return 0

            lax.fori_loop(0, nrows, row_body, 0)

        plsc.subcore_barrier()

        for d in (0, 1):
            def scat_fire(j, _, d=d):
                pltpu.async_copy(ones_v.at[0], hist_sh.at[keys.at[d * KR + j]],
                                 ssem, add=True)
                return 0

            lax.fori_loop(0, nrows, scat_fire, 0)

        for d in (0, 1):
            def scat_drain(j, _, d=d):
                pltpu.make_async_copy(ones_v.at[0],
                                      hist_sh.at[keys.at[d * KR + j]],
                                      ssem).wait()
                return 0

            lax.fori_loop(0, nrows, scat_drain, 0)

        plsc.subcore_barrier()

        ooff = pl.multiple_of(cid * HSIZE + zoff, 128)
        pltpu.sync_copy(hist_sh.at[pl.ds(zoff, CH)], out_hbm.at[pl.ds(ooff, CH)])

    return hist_kernel


@functools.lru_cache(maxsize=None)
def _build_combine_kernel(N: int, D: int, K: int, L: int, HROWS: int, BLK: int):
    grid = N // BLK
    assert grid * BLK == N and BLK % 8 == 0

    def body(c_ref, e_ref, w_ref, b_ref, o_ref):
        cb = c_ref[0] + c_ref[1]
        acc = jnp.dot(cb, b_ref[0],
                      preferred_element_type=jnp.float32)
        for k in range(K):
            p_k = jnp.dot(e_ref[...], w_ref[0, k],
                          preferred_element_type=jnp.float32)
            acc = acc + cb[:, k:k + 1] * p_k
        tot = jnp.sum(cb, axis=1, keepdims=True)
        o_ref[...] = acc / jnp.maximum(tot, 1.0)

    return pl.pallas_call(
        body,
        grid=(grid,),
        in_specs=[
            pl.BlockSpec((_NC, BLK, K), lambda i: (0, i, 0)),
            pl.BlockSpec((BLK, D), lambda i: (i, 0)),
            pl.BlockSpec((1, K, D, D), lambda i: (L - 1, 0, 0, 0)),
            pl.BlockSpec((1, K, D), lambda i: (L - 1, 0, 0)),
        ],
        out_specs=pl.BlockSpec((BLK, D), lambda i: (i, 0)),
        out_shape=jax.ShapeDtypeStruct((N, D), jnp.float32),
    )


def kernel(edge_index, edge_type, embeddings, W, b):
    N, D = embeddings.shape
    E = edge_index.shape[1]
    L = W.shape[0]
    K = W.shape[1]
    NBINS = N * K
    _, RB, _, _, HSIZE = _hist_sizes(E, K, NBINS)
    CH = HSIZE // _NS

    edge_flat = edge_index.reshape(2 * E)
    zeros = jnp.zeros((CH,), jnp.float32)
    ones = jnp.ones((RB, 128), jnp.float32)

    hist_fn = _build_hist_kernel(E, K, NBINS)
    flat = hist_fn(edge_flat, edge_type, zeros, ones)
    C3 = flat.reshape(_NC, HSIZE // K, K)

    out = _build_combine_kernel(N, D, K, L, HSIZE // K, 1000)(
        C3, embeddings, W, b)
    return out


# speedup vs baseline: 1.0924x; 1.0924x over previous
import functools

import jax
import jax.numpy as jnp
from jax import lax
from jax.experimental import pallas as pl
from jax.experimental.pallas import tpu as pltpu
from jax.experimental.pallas import tpu_sc as plsc

_NC = 2
_NS = 16
_LN = 16


def _hist_sizes(E: int, K: int, NBINS: int):
    NW = _NC * _NS
    ROWS = E // 128
    RB = ROWS // NW
    EXTRA = ROWS - RB * NW
    KR = RB + (1 if EXTRA else 0)
    HSIZE = -(-NBINS // (_NS * 128)) * (_NS * 128)
    return ROWS, RB, EXTRA, KR, HSIZE


@functools.lru_cache(maxsize=None)
def _build_hist_kernel(E: int, K: int, NBINS: int):
    NW = _NC * _NS
    R = K // 2
    ROWS, RB, EXTRA, KR, HSIZE = _hist_sizes(E, K, NBINS)
    EPW = KR * 128
    CH = HSIZE // _NS
    HROWS = HSIZE // K

    mesh = plsc.VectorSubcoreMesh(
        core_axis_name="c", subcore_axis_name="s",
        num_cores=_NC, num_subcores=_NS)

    @functools.partial(
        pl.kernel,
        out_type=jax.ShapeDtypeStruct((_NC * HSIZE,), jnp.float32),
        mesh=mesh,
        scratch_types=[
            pltpu.VMEM((EPW,), jnp.int32),
            pltpu.VMEM((EPW,), jnp.int32),
            pltpu.VMEM((2 * KR, 128), jnp.int32),
            pltpu.VMEM((RB, 128), jnp.float32),
            pltpu.VMEM_SHARED((HSIZE,), jnp.float32),
            pltpu.SemaphoreType.DMA,
        ],
    )
    def hist_kernel(ei_hbm, et_hbm, zeros_hbm, ones_hbm, out_hbm,
                    tbuf, ebuf, keys, ones_v, hist_sh, ssem):
        cid = lax.axis_index("c")
        sid = lax.axis_index("s")
        wid = cid * _NS + sid
        nrows = RB + jnp.where(wid < EXTRA, 1, 0)
        row0 = wid * RB + jnp.minimum(wid, EXTRA)
        base = pl.multiple_of(row0 * 128, 128)
        zoff = pl.multiple_of(sid * CH, 128)

        pltpu.sync_copy(zeros_hbm, hist_sh.at[pl.ds(zoff, CH)])
        pltpu.sync_copy(ones_hbm, ones_v)
        pltpu.sync_copy(et_hbm.at[pl.ds(base, RB * 128)], tbuf.at[pl.ds(0, RB * 128)])
        if EXTRA:
            @pl.when(wid < EXTRA)
            def _():
                off = pl.multiple_of(base + RB * 128, 128)
                pltpu.sync_copy(et_hbm.at[pl.ds(off, 128)],
                                tbuf.at[pl.ds(RB * 128, 128)])

        first_barrier = [True]
        for d in (0, 1):
            dbase = pl.multiple_of(d * E + base, 128)
            pltpu.sync_copy(ei_hbm.at[pl.ds(dbase, RB * 128)],
                            ebuf.at[pl.ds(0, RB * 128)])
            if EXTRA:
                @pl.when(wid < EXTRA)
                def _(d=d, dbase=dbase):
                    off = pl.multiple_of(dbase + RB * 128, 128)
                    pltpu.sync_copy(ei_hbm.at[pl.ds(off, 128)],
                                    ebuf.at[pl.ds(RB * 128, 128)])

            def row_body(j, _, d=d):
                for c in range(8):
                    off = pl.multiple_of(j * 128 + c * _LN, 8)
                    vi = ebuf[pl.ds(off, _LN)]
                    vt = tbuf[pl.ds(off, _LN)]
                    keys[d * KR + j, pl.ds(c * _LN, _LN)] = (
                        (vt + d * R) * HROWS + vi)
                return 0

            lax.fori_loop(0, nrows, row_body, 0)

            if first_barrier[0]:
                plsc.subcore_barrier()
                first_barrier[0] = False

            def scat_fire(j, _, d=d):
                pltpu.async_copy(ones_v.at[0], hist_sh.at[keys.at[d * KR + j]],
                                 ssem, add=True)
                return 0

            lax.fori_loop(0, nrows, scat_fire, 0)

        for d in (0, 1):
            def scat_drain(j, _, d=d):
                pltpu.make_async_copy(ones_v.at[0],
                                      hist_sh.at[keys.at[d * KR + j]],
                                      ssem).wait()
                return 0

            lax.fori_loop(0, nrows, scat_drain, 0)

        plsc.subcore_barrier()

        ooff = pl.multiple_of(cid * HSIZE + zoff, 128)
        pltpu.sync_copy(hist_sh.at[pl.ds(zoff, CH)], out_hbm.at[pl.ds(ooff, CH)])

    return hist_kernel


@functools.lru_cache(maxsize=None)
def _build_combine_kernel(N: int, D: int, K: int, L: int, HROWS: int):
    def body(c_ref, e_ref, w_ref, b_ref, o_ref):
        cbt = c_ref[0] + c_ref[1]
        cb = jnp.transpose(cbt)[:N]
        acc = jnp.dot(cb, b_ref[0],
                      preferred_element_type=jnp.float32)
        for k in range(K):
            p_k = jnp.dot(e_ref[...], w_ref[0, k],
                          preferred_element_type=jnp.float32)
            acc = acc + cb[:, k:k + 1] * p_k
        tot = jnp.sum(cb, axis=1, keepdims=True)
        o_ref[...] = acc / jnp.maximum(tot, 1.0)

    return pl.pallas_call(
        body,
        grid=(1,),
        in_specs=[
            pl.BlockSpec((_NC, K, HROWS), lambda i: (0, 0, 0)),
            pl.BlockSpec((N, D), lambda i: (0, 0)),
            pl.BlockSpec((1, K, D, D), lambda i: (L - 1, 0, 0, 0)),
            pl.BlockSpec((1, K, D), lambda i: (L - 1, 0, 0)),
        ],
        out_specs=pl.BlockSpec((N, D), lambda i: (0, 0)),
        out_shape=jax.ShapeDtypeStruct((N, D), jnp.float32),
    )


def kernel(edge_index, edge_type, embeddings, W, b):
    N, D = embeddings.shape
    E = edge_index.shape[1]
    L = W.shape[0]
    K = W.shape[1]
    NBINS = N * K
    _, RB, _, _, HSIZE = _hist_sizes(E, K, NBINS)
    CH = HSIZE // _NS

    edge_flat = edge_index.reshape(2 * E)
    zeros = jnp.zeros((CH,), jnp.float32)
    ones = jnp.ones((RB, 128), jnp.float32)

    hist_fn = _build_hist_kernel(E, K, NBINS)
    flat = hist_fn(edge_flat, edge_type, zeros, ones)
    C3 = flat.reshape(_NC, K, HSIZE // K)

    out = _build_combine_kernel(N, D, K, L, HSIZE // K)(
        C3, embeddings, W, b)
    return out
